# baseline (speedup 1.0000x reference)
import functools

import jax
import jax.numpy as jnp
from jax import lax
from jax.experimental import pallas as pl
from jax.experimental.pallas import tpu as pltpu
from jax.experimental.pallas import tpu_sc as plsc

VOCAB = 1000000
EMBED = 64
HIDDEN = 64
B = 1024
T = 200

NUM_CORES = 2
NUM_SUBCORES = 16
NUM_WORKERS = NUM_CORES * NUM_SUBCORES

GATHER_CHUNK = 128

S_PER_BLOCK = 8


def _sc_gather(table, idx_flat):
    n = idx_flat.shape[0]
    per_w = n // NUM_WORKERS
    chunks = per_w // GATHER_CHUNK
    assert per_w * NUM_WORKERS == n and chunks * GATHER_CHUNK == per_w

    mesh = plsc.VectorSubcoreMesh(core_axis_name="c", subcore_axis_name="s")

    @functools.partial(
        pl.kernel,
        out_type=jax.ShapeDtypeStruct((n, 2 * EMBED), jnp.float32),
        mesh=mesh,
        scratch_types=[
            pltpu.VMEM((GATHER_CHUNK,), jnp.int32),
            pltpu.VMEM((GATHER_CHUNK, EMBED), jnp.float32),
            pltpu.SemaphoreType.DMA,
        ],
        compiler_params=pltpu.CompilerParams(use_tc_tiling_on_sc=False),
    )
    def gather_kernel(table_hbm, idx_hbm, out_hbm, idx_v, rows_v, sem):
        wid = lax.axis_index("s") * NUM_CORES + lax.axis_index("c")
        base_w = wid * per_w

        @pl.loop(0, chunks)
        def _chunk(c):
            base = base_w + c * GATHER_CHUNK
            pltpu.sync_copy(idx_hbm.at[pl.ds(base, GATHER_CHUNK)], idx_v)
            pltpu.async_copy(table_hbm.at[idx_v], rows_v, sem).wait()
            pltpu.sync_copy(
                rows_v, out_hbm.at[pl.ds(base, GATHER_CHUNK), pl.ds(0, EMBED)]
            )

    return gather_kernel(table, idx_flat)


def _lstm_body(xs_ref, wx_ref, wh_ref, b_ref, ys_ref, h_ref, c_ref):
    gi = pl.program_id(0)

    @pl.when(gi == 0)
    def _init():
        h_ref[...] = jnp.zeros_like(h_ref)
        c_ref[...] = jnp.zeros_like(c_ref)

    h = h_ref[...]
    c = c_ref[...]
    wx = wx_ref[...]
    wh = wh_ref[...]
    bias = b_ref[...]
    for k in range(S_PER_BLOCK):
        x = xs_ref[:, k, 0:EMBED]
        gates = (jnp.dot(x, wx, preferred_element_type=jnp.float32)
                 + jnp.dot(h, wh, preferred_element_type=jnp.float32)
                 + bias)
        gi_ = jax.nn.sigmoid(gates[:, 0 * HIDDEN:1 * HIDDEN])
        gf = jax.nn.sigmoid(gates[:, 1 * HIDDEN:2 * HIDDEN])
        gg = jnp.tanh(gates[:, 2 * HIDDEN:3 * HIDDEN])
        go = jax.nn.sigmoid(gates[:, 3 * HIDDEN:4 * HIDDEN])
        c = gf * c + gi_ * gg
        h = go * jnp.tanh(c)
        ys_ref[:, k, :] = h
    h_ref[...] = h
    c_ref[...] = c


def _tc_lstm(xs3, wx, wh, bias):
    nblk = T // S_PER_BLOCK
    return pl.pallas_call(
        _lstm_body,
        grid=(nblk,),
        in_specs=[
            pl.BlockSpec((B, S_PER_BLOCK, 2 * EMBED), lambda i: (0, i, 0)),
            pl.BlockSpec((EMBED, 4 * HIDDEN), lambda i: (0, 0)),
            pl.BlockSpec((HIDDEN, 4 * HIDDEN), lambda i: (0, 0)),
            pl.BlockSpec((1, 4 * HIDDEN), lambda i: (0, 0)),
        ],
        out_specs=[
            pl.BlockSpec((B, S_PER_BLOCK, HIDDEN), lambda i: (0, i, 0)),
            pl.BlockSpec((B, HIDDEN), lambda i: (0, 0)),
            pl.BlockSpec((B, HIDDEN), lambda i: (0, 0)),
        ],
        out_shape=[
            jax.ShapeDtypeStruct((B, T, HIDDEN), jnp.float32),
            jax.ShapeDtypeStruct((B, HIDDEN), jnp.float32),
            jax.ShapeDtypeStruct((B, HIDDEN), jnp.float32),
        ],
        compiler_params=pltpu.CompilerParams(
            dimension_semantics=("arbitrary",),
        ),
    )(xs3, wx, wh, bias)


def kernel(src, table, W_ih, W_hh, b_ih, b_hh):
    idx_flat = src.reshape(-1)
    o128 = _sc_gather(table, idx_flat)
    xs3 = o128.reshape(B, T, 2 * EMBED)

    wx = jnp.transpose(W_ih)
    wh = jnp.transpose(W_hh)
    bias = (b_ih + b_hh)[None, :]

    ys, hT, cT = _tc_lstm(xs3, wx, wh, bias)
    return (ys, hT[None, :, :], cT[None, :, :])


# speedup vs baseline: 1.2724x; 1.2724x over previous
import functools

import jax
import jax.numpy as jnp
from jax import lax
from jax.experimental import pallas as pl
from jax.experimental.pallas import tpu as pltpu
from jax.experimental.pallas import tpu_sc as plsc

VOCAB = 1000000
EMBED = 64
HIDDEN = 64
B = 1024
T = 200

NUM_CORES = 2
NUM_SUBCORES = 16
NUM_WORKERS = NUM_CORES * NUM_SUBCORES
LANES = 16

GATHER_CHUNK = 128

S_PER_BLOCK = 8


def _sc_gather(table, idx_flat):
    n = idx_flat.shape[0]
    per_w = n // NUM_WORKERS
    chunks = per_w // GATHER_CHUNK
    assert per_w * NUM_WORKERS == n and chunks * GATHER_CHUNK == per_w

    mesh = plsc.VectorSubcoreMesh(core_axis_name="c", subcore_axis_name="s")

    @functools.partial(
        pl.kernel,
        out_type=jax.ShapeDtypeStruct((n, 2 * EMBED), jnp.float32),
        mesh=mesh,
        scratch_types=[
            pltpu.VMEM((GATHER_CHUNK,), jnp.int32),
            pltpu.VMEM((GATHER_CHUNK,), jnp.int32),
            pltpu.VMEM((GATHER_CHUNK, EMBED), jnp.float32),
            pltpu.SemaphoreType.DMA,
            pltpu.SemaphoreType.DMA,
        ],
        compiler_params=pltpu.CompilerParams(use_tc_tiling_on_sc=False),
    )
    def gather_kernel(table_hbm, idx_hbm, out_hbm, pos_v, idxg_v, rows_v,
                      sem_i, sem_r):
        wid = lax.axis_index("s") * NUM_CORES + lax.axis_index("c")
        base_w = wid * per_w
        lane = lax.iota(jnp.int32, LANES)

        @pl.loop(0, chunks)
        def _chunk(c):
            base = base_w + c * GATHER_CHUNK
            for j in range(GATHER_CHUNK // LANES):
                q = base + j * LANES + lane
                t = lax.shift_right_logical(q, 10)
                b = lax.bitwise_and(q, B - 1)
                pos_v[pl.ds(j * LANES, LANES)] = b * T + t
            pltpu.async_copy(idx_hbm.at[pos_v], idxg_v, sem_i).wait()
            pltpu.async_copy(table_hbm.at[idxg_v], rows_v, sem_r).wait()
            pltpu.sync_copy(
                rows_v, out_hbm.at[pl.ds(base, GATHER_CHUNK), pl.ds(0, EMBED)]
            )

    return gather_kernel(table, idx_flat)


def _lstm_body(xs_ref, wx_ref, wh_ref, b_ref, ys_ref, h_ref, c_ref):
    gi = pl.program_id(0)

    @pl.when(gi == 0)
    def _init():
        h_ref[...] = jnp.zeros_like(h_ref)
        c_ref[...] = jnp.zeros_like(c_ref)

    h = h_ref[...]
    c = c_ref[...]
    wx = wx_ref[...]
    wh = wh_ref[...]
    bias = b_ref[...]
    for k in range(S_PER_BLOCK):
        x = xs_ref[k * B:(k + 1) * B, 0:EMBED]
        gates = (jnp.dot(x, wx, preferred_element_type=jnp.float32)
                 + jnp.dot(h, wh, preferred_element_type=jnp.float32)
                 + bias)
        gi_ = jax.nn.sigmoid(gates[:, 0 * HIDDEN:1 * HIDDEN])
        gf = jax.nn.sigmoid(gates[:, 1 * HIDDEN:2 * HIDDEN])
        gg = jnp.tanh(gates[:, 2 * HIDDEN:3 * HIDDEN])
        go = jax.nn.sigmoid(gates[:, 3 * HIDDEN:4 * HIDDEN])
        c = gf * c + gi_ * gg
        h = go * jnp.tanh(c)
        ys_ref[:, k * HIDDEN:(k + 1) * HIDDEN] = h
    h_ref[...] = h
    c_ref[...] = c


def _tc_lstm(xs, wx, wh, bias):
    nblk = T // S_PER_BLOCK
    return pl.pallas_call(
        _lstm_body,
        grid=(nblk,),
        in_specs=[
            pl.BlockSpec((S_PER_BLOCK * B, 2 * EMBED), lambda i: (i, 0)),
            pl.BlockSpec((EMBED, 4 * HIDDEN), lambda i: (0, 0)),
            pl.BlockSpec((HIDDEN, 4 * HIDDEN), lambda i: (0, 0)),
            pl.BlockSpec((1, 4 * HIDDEN), lambda i: (0, 0)),
        ],
        out_specs=[
            pl.BlockSpec((B, S_PER_BLOCK * HIDDEN), lambda i: (0, i)),
            pl.BlockSpec((B, HIDDEN), lambda i: (0, 0)),
            pl.BlockSpec((B, HIDDEN), lambda i: (0, 0)),
        ],
        out_shape=[
            jax.ShapeDtypeStruct((B, T * HIDDEN), jnp.float32),
            jax.ShapeDtypeStruct((B, HIDDEN), jnp.float32),
            jax.ShapeDtypeStruct((B, HIDDEN), jnp.float32),
        ],
        compiler_params=pltpu.CompilerParams(
            dimension_semantics=("arbitrary",),
        ),
    )(xs, wx, wh, bias)


def kernel(src, table, W_ih, W_hh, b_ih, b_hh):
    idx_flat = src.reshape(-1)
    o128 = _sc_gather(table, idx_flat)

    wx = jnp.transpose(W_ih)
    wh = jnp.transpose(W_hh)
    bias = (b_ih + b_hh)[None, :]

    ys, hT, cT = _tc_lstm(o128, wx, wh, bias)
    outputs = ys.reshape(B, T, HIDDEN)
    return (outputs, hT[None, :, :], cT[None, :, :])


# speedup vs baseline: 1.2733x; 1.0007x over previous
import functools

import jax
import jax.numpy as jnp
from jax import lax
from jax.experimental import pallas as pl
from jax.experimental.pallas import tpu as pltpu
from jax.experimental.pallas import tpu_sc as plsc

VOCAB = 1000000
EMBED = 64
HIDDEN = 64
B = 1024
T = 200

NUM_CORES = 2
NUM_SUBCORES = 16
NUM_WORKERS = NUM_CORES * NUM_SUBCORES
LANES = 16

GATHER_CHUNK = 128

S_PER_BLOCK = 8


def _sc_gather(table, idx_flat):
    n = idx_flat.shape[0]
    per_w = n // NUM_WORKERS
    chunks = per_w // GATHER_CHUNK
    assert per_w * NUM_WORKERS == n and chunks * GATHER_CHUNK == per_w

    mesh = plsc.VectorSubcoreMesh(core_axis_name="c", subcore_axis_name="s")

    @functools.partial(
        pl.kernel,
        out_type=jax.ShapeDtypeStruct((n, 2 * EMBED), jnp.float32),
        mesh=mesh,
        scratch_types=[
            pltpu.VMEM((GATHER_CHUNK,), jnp.int32),
            pltpu.VMEM((GATHER_CHUNK,), jnp.int32),
            pltpu.VMEM((GATHER_CHUNK, EMBED), jnp.float32),
            pltpu.SemaphoreType.DMA,
            pltpu.SemaphoreType.DMA,
        ],
        compiler_params=pltpu.CompilerParams(use_tc_tiling_on_sc=False),
    )
    def gather_kernel(table_hbm, idx_hbm, out_hbm, pos_v, idxg_v, rows_v,
                      sem_i, sem_r):
        wid = lax.axis_index("s") * NUM_CORES + lax.axis_index("c")
        base_w = wid * per_w
        lane = lax.iota(jnp.int32, LANES)

        @pl.loop(0, chunks)
        def _chunk(c):
            base = base_w + c * GATHER_CHUNK
            for j in range(GATHER_CHUNK // LANES):
                q = base + j * LANES + lane
                t = lax.shift_right_logical(q, 10)
                b = lax.bitwise_and(q, B - 1)
                pos_v[pl.ds(j * LANES, LANES)] = b * T + t
            pltpu.async_copy(idx_hbm.at[pos_v], idxg_v, sem_i).wait()
            pltpu.async_copy(table_hbm.at[idxg_v], rows_v, sem_r).wait()
            pltpu.sync_copy(
                rows_v, out_hbm.at[pl.ds(base, GATHER_CHUNK), pl.ds(0, EMBED)]
            )

    return gather_kernel(table, idx_flat)


def _lstm_body(xs_ref, wx_ref, wh_ref, b_ref, ys_ref, h_ref, c_ref):
    gi = pl.program_id(0)

    @pl.when(gi == 0)
    def _init():
        h_ref[...] = jnp.zeros_like(h_ref)
        c_ref[...] = jnp.zeros_like(c_ref)

    h = h_ref[...]
    c = c_ref[...]
    wx = wx_ref[...]
    wh = wh_ref[...]
    bias = b_ref[...]
    for k in range(S_PER_BLOCK):
        x = xs_ref[k * B:(k + 1) * B, 0:EMBED]
        gates = (jnp.dot(x, wx, preferred_element_type=jnp.float32)
                 + jnp.dot(h, wh, preferred_element_type=jnp.float32)
                 + bias)
        gi_ = jax.nn.sigmoid(gates[:, 0 * HIDDEN:1 * HIDDEN])
        gf = jax.nn.sigmoid(gates[:, 1 * HIDDEN:2 * HIDDEN])
        gg = jnp.tanh(gates[:, 2 * HIDDEN:3 * HIDDEN])
        go = jax.nn.sigmoid(gates[:, 3 * HIDDEN:4 * HIDDEN])
        c = gf * c + gi_ * gg
        h = go * jnp.tanh(c)
        ys_ref[:, k * HIDDEN:(k + 1) * HIDDEN] = h
    h_ref[...] = h
    c_ref[...] = c


def _tc_lstm(xs, wx, wh, bias):
    nblk = T // S_PER_BLOCK
    return pl.pallas_call(
        _lstm_body,
        grid=(nblk,),
        in_specs=[
            pl.BlockSpec((S_PER_BLOCK * B, 2 * EMBED), lambda i: (i, 0)),
            pl.BlockSpec((EMBED, 4 * HIDDEN), lambda i: (0, 0)),
            pl.BlockSpec((HIDDEN, 4 * HIDDEN), lambda i: (0, 0)),
            pl.BlockSpec((1, 4 * HIDDEN), lambda i: (0, 0)),
        ],
        out_specs=[
            pl.BlockSpec((B, S_PER_BLOCK * HIDDEN), lambda i: (0, i)),
            pl.BlockSpec((B, HIDDEN), lambda i: (0, 0)),
            pl.BlockSpec((B, HIDDEN), lambda i: (0, 0)),
        ],
        out_shape=[
            jax.ShapeDtypeStruct((B, T * HIDDEN), jnp.float32),
            jax.ShapeDtypeStruct((B, HIDDEN), jnp.float32),
            jax.ShapeDtypeStruct((B, HIDDEN), jnp.float32),
        ],
        compiler_params=pltpu.CompilerParams(
            dimension_semantics=("arbitrary",),
            allow_input_fusion=[True, False, False, False],
        ),
    )(xs, wx, wh, bias)


def kernel(src, table, W_ih, W_hh, b_ih, b_hh):
    idx_flat = src.reshape(-1)
    o128 = _sc_gather(table, idx_flat)

    wx = jnp.transpose(W_ih)
    wh = jnp.transpose(W_hh)
    bias = (b_ih + b_hh)[None, :]

    ys, hT, cT = _tc_lstm(o128, wx, wh, bias)
    outputs = ys.reshape(B, T, HIDDEN)
    return (outputs, hT[None, :, :], cT[None, :, :])


# speedup vs baseline: 1.3096x; 1.0285x over previous
import functools

import jax
import jax.numpy as jnp
from jax import lax
from jax.experimental import pallas as pl
from jax.experimental.pallas import tpu as pltpu
from jax.experimental.pallas import tpu_sc as plsc

VOCAB = 1000000
EMBED = 64
HIDDEN = 64
B = 1024
T = 200

NUM_CORES = 2
NUM_SUBCORES = 16
NUM_WORKERS = NUM_CORES * NUM_SUBCORES
LANES = 16

GATHER_CHUNK = 128

S_PER_BLOCK = 8


def _sc_gather(table, idx_flat):
    n = idx_flat.shape[0]
    per_w = n // NUM_WORKERS
    chunks = per_w // GATHER_CHUNK
    assert per_w * NUM_WORKERS == n and chunks * GATHER_CHUNK == per_w

    mesh = plsc.VectorSubcoreMesh(core_axis_name="c", subcore_axis_name="s")

    @functools.partial(
        pl.kernel,
        out_type=jax.ShapeDtypeStruct((n, 2 * EMBED), jnp.float32),
        mesh=mesh,
        scratch_types=[
            pltpu.VMEM((GATHER_CHUNK,), jnp.int32),
            pltpu.VMEM((GATHER_CHUNK,), jnp.int32),
            pltpu.VMEM((GATHER_CHUNK, 2 * EMBED), jnp.float32),
            pltpu.SemaphoreType.DMA,
            pltpu.SemaphoreType.DMA,
        ],
        compiler_params=pltpu.CompilerParams(use_tc_tiling_on_sc=False),
    )
    def gather_kernel(table_hbm, idx_hbm, out_hbm, pos_v, idxg_v, rows_v,
                      sem_i, sem_r):
        wid = lax.axis_index("s") * NUM_CORES + lax.axis_index("c")
        base_w = wid * per_w
        lane = lax.iota(jnp.int32, LANES)

        @pl.loop(0, chunks)
        def _chunk(c):
            base = base_w + c * GATHER_CHUNK
            for j in range(GATHER_CHUNK // LANES):
                q = base + j * LANES + lane
                t = lax.shift_right_logical(q, 10)
                b = lax.bitwise_and(q, B - 1)
                pos_v[pl.ds(j * LANES, LANES)] = b * T + t
            pltpu.async_copy(idx_hbm.at[pos_v], idxg_v, sem_i).wait()
            pltpu.async_copy(table_hbm.at[idxg_v], rows_v, sem_r).wait()
            pltpu.sync_copy(rows_v, out_hbm.at[pl.ds(base, GATHER_CHUNK)])

    return gather_kernel(table, idx_flat)


def _lstm_body(xs_ref, wx_ref, wh_ref, b_ref, ys_ref, h_ref, c_ref):
    gi = pl.program_id(0)

    @pl.when(gi == 0)
    def _init():
        h_ref[...] = jnp.zeros_like(h_ref)
        c_ref[...] = jnp.zeros_like(c_ref)

    h = h_ref[...]
    c = c_ref[...]
    wx = wx_ref[...]
    wh = wh_ref[...]
    bias = b_ref[...]
    for k in range(S_PER_BLOCK):
        x = xs_ref[k * B:(k + 1) * B, 0:EMBED]
        gates = (jnp.dot(x, wx, preferred_element_type=jnp.float32)
                 + jnp.dot(h, wh, preferred_element_type=jnp.float32)
                 + bias)
        gi_ = jax.nn.sigmoid(gates[:, 0 * HIDDEN:1 * HIDDEN])
        gf = jax.nn.sigmoid(gates[:, 1 * HIDDEN:2 * HIDDEN])
        gg = jnp.tanh(gates[:, 2 * HIDDEN:3 * HIDDEN])
        go = jax.nn.sigmoid(gates[:, 3 * HIDDEN:4 * HIDDEN])
        c = gf * c + gi_ * gg
        h = go * jnp.tanh(c)
        ys_ref[:, k * HIDDEN:(k + 1) * HIDDEN] = h
    h_ref[...] = h
    c_ref[...] = c


def _tc_lstm(xs, wx, wh, bias):
    nblk = T // S_PER_BLOCK
    return pl.pallas_call(
        _lstm_body,
        grid=(nblk,),
        in_specs=[
            pl.BlockSpec((S_PER_BLOCK * B, 2 * EMBED), lambda i: (i, 0)),
            pl.BlockSpec((EMBED, 4 * HIDDEN), lambda i: (0, 0)),
            pl.BlockSpec((HIDDEN, 4 * HIDDEN), lambda i: (0, 0)),
            pl.BlockSpec((1, 4 * HIDDEN), lambda i: (0, 0)),
        ],
        out_specs=[
            pl.BlockSpec((B, S_PER_BLOCK * HIDDEN), lambda i: (0, i)),
            pl.BlockSpec((B, HIDDEN), lambda i: (0, 0)),
            pl.BlockSpec((B, HIDDEN), lambda i: (0, 0)),
        ],
        out_shape=[
            jax.ShapeDtypeStruct((B, T * HIDDEN), jnp.float32),
            jax.ShapeDtypeStruct((B, HIDDEN), jnp.float32),
            jax.ShapeDtypeStruct((B, HIDDEN), jnp.float32),
        ],
        compiler_params=pltpu.CompilerParams(
            dimension_semantics=("arbitrary",),
            allow_input_fusion=[True, False, False, False],
        ),
    )(xs, wx, wh, bias)


def kernel(src, table, W_ih, W_hh, b_ih, b_hh):
    idx_flat = src.reshape(-1)
    t128 = jnp.pad(table, ((0, 0), (0, EMBED)))
    o128 = _sc_gather(t128, idx_flat)

    wx = jnp.transpose(W_ih)
    wh = jnp.transpose(W_hh)
    bias = (b_ih + b_hh)[None, :]

    ys, hT, cT = _tc_lstm(o128, wx, wh, bias)
    outputs = ys.reshape(B, T, HIDDEN)
    return (outputs, hT[None, :, :], cT[None, :, :])


# speedup vs baseline: 1.5149x; 1.1568x over previous
import functools

import jax
import jax.numpy as jnp
from jax import lax
from jax.experimental import pallas as pl
from jax.experimental.pallas import tpu as pltpu
from jax.experimental.pallas import tpu_sc as plsc

VOCAB = 1000000
EMBED = 64
HIDDEN = 64
B = 1024
T = 200

NUM_CORES = 2
NUM_SUBCORES = 16
NUM_WORKERS = NUM_CORES * NUM_SUBCORES
LANES = 16

GATHER_CHUNK = 128

S_PER_BLOCK = 8


def _sc_gather(table, idx_flat):
    n = idx_flat.shape[0]
    per_w = n // NUM_WORKERS
    chunks = per_w // GATHER_CHUNK
    assert per_w * NUM_WORKERS == n and chunks * GATHER_CHUNK == per_w
    assert chunks % 2 == 0

    mesh = plsc.VectorSubcoreMesh(core_axis_name="c", subcore_axis_name="s")

    @functools.partial(
        pl.kernel,
        out_type=jax.ShapeDtypeStruct((n, 2 * EMBED), jnp.float32),
        mesh=mesh,
        scratch_types=[
            pltpu.VMEM((2, GATHER_CHUNK), jnp.int32),
            pltpu.VMEM((2, GATHER_CHUNK), jnp.int32),
            pltpu.VMEM((2, GATHER_CHUNK, 2 * EMBED), jnp.float32),
            pltpu.SemaphoreType.DMA((2,)),
            pltpu.SemaphoreType.DMA((2,)),
        ],
        compiler_params=pltpu.CompilerParams(use_tc_tiling_on_sc=False),
    )
    def gather_kernel(table_hbm, idx_hbm, out_hbm, pos_v, idxg_v, rows_v,
                      sem_i, sem_r):
        wid = lax.axis_index("s") * NUM_CORES + lax.axis_index("c")
        base_w = wid * per_w
        lane = lax.iota(jnp.int32, LANES)

        def fire_idx(c, buf):
            base = base_w + c * GATHER_CHUNK
            for j in range(GATHER_CHUNK // LANES):
                q = base + j * LANES + lane
                t = lax.shift_right_logical(q, 10)
                b = lax.bitwise_and(q, B - 1)
                pos_v[buf, pl.ds(j * LANES, LANES)] = b * T + t
            pltpu.async_copy(idx_hbm.at[pos_v.at[buf]], idxg_v.at[buf],
                             sem_i.at[buf])

        def wait_idx(buf):
            pltpu.make_async_copy(idx_hbm.at[pl.ds(0, GATHER_CHUNK)],
                                  idxg_v.at[buf], sem_i.at[buf]).wait()

        def fire_rows(buf):
            pltpu.async_copy(table_hbm.at[idxg_v.at[buf]], rows_v.at[buf],
                             sem_r.at[buf])

        def wait_rows(buf):
            pltpu.make_async_copy(table_hbm.at[pl.ds(0, GATHER_CHUNK)],
                                  rows_v.at[buf], sem_r.at[buf]).wait()

        def write_out(c, buf):
            base = base_w + c * GATHER_CHUNK
            pltpu.sync_copy(rows_v.at[buf],
                            out_hbm.at[pl.ds(base, GATHER_CHUNK)])

        fire_idx(0, 0)
        wait_idx(0)
        fire_rows(0)
        fire_idx(1, 1)

        @pl.loop(0, chunks, step=2)
        def _chunk(c):
            wait_idx(1)
            fire_rows(1)
            wait_rows(0)
            write_out(c, 0)

            @pl.when(c + 2 < chunks)
            def _next_even():
                fire_idx(c + 2, 0)
                wait_idx(0)
                fire_rows(0)

            wait_rows(1)
            write_out(c + 1, 1)

            @pl.when(c + 3 < chunks)
            def _next_odd():
                fire_idx(c + 3, 1)

    return gather_kernel(table, idx_flat)


def _lstm_body(xs_ref, wi_ref, wh_ref, b_ref, ys_ref, h_ref, c_ref):
    gi = pl.program_id(0)

    @pl.when(gi == 0)
    def _init():
        h_ref[...] = jnp.zeros_like(h_ref)
        c_ref[...] = jnp.zeros_like(c_ref)

    h = h_ref[...]
    c = c_ref[...]
    wi = wi_ref[...]
    wh = wh_ref[...]
    bias = b_ref[...]
    for k in range(S_PER_BLOCK):
        x = xs_ref[k * B:(k + 1) * B, 0:EMBED]
        gates = (
            lax.dot_general(wi, x, (((1,), (1,)), ((), ())),
                            preferred_element_type=jnp.float32)
            + lax.dot_general(wh, h, (((1,), (0,)), ((), ())),
                              preferred_element_type=jnp.float32)
            + bias
        )
        gi_ = jax.nn.sigmoid(gates[0 * HIDDEN:1 * HIDDEN, :])
        gf = jax.nn.sigmoid(gates[1 * HIDDEN:2 * HIDDEN, :])
        gg = jnp.tanh(gates[2 * HIDDEN:3 * HIDDEN, :])
        go = jax.nn.sigmoid(gates[3 * HIDDEN:4 * HIDDEN, :])
        c = gf * c + gi_ * gg
        h = go * jnp.tanh(c)
        ys_ref[k * HIDDEN:(k + 1) * HIDDEN, :] = h
    h_ref[...] = h
    c_ref[...] = c


def _tc_lstm(xs, wi, wh, bias):
    nblk = T // S_PER_BLOCK
    return pl.pallas_call(
        _lstm_body,
        grid=(nblk,),
        in_specs=[
            pl.BlockSpec((S_PER_BLOCK * B, 2 * EMBED), lambda i: (i, 0)),
            pl.BlockSpec((4 * HIDDEN, EMBED), lambda i: (0, 0)),
            pl.BlockSpec((4 * HIDDEN, HIDDEN), lambda i: (0, 0)),
            pl.BlockSpec((4 * HIDDEN, 1), lambda i: (0, 0)),
        ],
        out_specs=[
            pl.BlockSpec((S_PER_BLOCK * HIDDEN, B), lambda i: (i, 0)),
            pl.BlockSpec((HIDDEN, B), lambda i: (0, 0)),
            pl.BlockSpec((HIDDEN, B), lambda i: (0, 0)),
        ],
        out_shape=[
            jax.ShapeDtypeStruct((T * HIDDEN, B), jnp.float32),
            jax.ShapeDtypeStruct((HIDDEN, B), jnp.float32),
            jax.ShapeDtypeStruct((HIDDEN, B), jnp.float32),
        ],
        compiler_params=pltpu.CompilerParams(
            dimension_semantics=("arbitrary",),
        ),
    )(xs, wi, wh, bias)


def kernel(src, table, W_ih, W_hh, b_ih, b_hh):
    idx_flat = src.reshape(-1)
    t128 = jnp.pad(table, ((0, 0), (0, EMBED)))
    o128 = _sc_gather(t128, idx_flat)

    bias = (b_ih + b_hh)[:, None]

    ysT, hT, cT = _tc_lstm(o128, W_ih, W_hh, bias)
    outputs = ysT.reshape(T, HIDDEN, B).transpose(2, 0, 1)
    hidden = hT.transpose(1, 0)[None]
    cell = cT.transpose(1, 0)[None]
    return (outputs, hidden, cell)


# speedup vs baseline: 1.5912x; 1.0504x over previous
import functools

import jax
import jax.numpy as jnp
from jax import lax
from jax.experimental import pallas as pl
from jax.experimental.pallas import tpu as pltpu
from jax.experimental.pallas import tpu_sc as plsc

VOCAB = 1000000
EMBED = 64
HIDDEN = 64
B = 1024
T = 200
T1 = 96
T2 = T - T1

NUM_CORES = 2
NUM_SUBCORES = 16
NUM_WORKERS = NUM_CORES * NUM_SUBCORES
LANES = 16

GATHER_CHUNK = 128

S_PER_BLOCK = 8


def _sc_gather(table, idx_flat, q0, n):
    per_w = n // NUM_WORKERS
    chunks = per_w // GATHER_CHUNK
    assert per_w * NUM_WORKERS == n and chunks * GATHER_CHUNK == per_w
    assert chunks % 2 == 0 and q0 % GATHER_CHUNK == 0

    mesh = plsc.VectorSubcoreMesh(core_axis_name="c", subcore_axis_name="s")

    @functools.partial(
        pl.kernel,
        out_type=jax.ShapeDtypeStruct((n, 2 * EMBED), jnp.float32),
        mesh=mesh,
        scratch_types=[
            pltpu.VMEM((2, GATHER_CHUNK), jnp.int32),
            pltpu.VMEM((2, GATHER_CHUNK), jnp.int32),
            pltpu.VMEM((2, GATHER_CHUNK, 2 * EMBED), jnp.float32),
            pltpu.SemaphoreType.DMA((2,)),
            pltpu.SemaphoreType.DMA((2,)),
        ],
        compiler_params=pltpu.CompilerParams(use_tc_tiling_on_sc=False),
    )
    def gather_kernel(table_hbm, idx_hbm, out_hbm, pos_v, idxg_v, rows_v,
                      sem_i, sem_r):
        wid = lax.axis_index("s") * NUM_CORES + lax.axis_index("c")
        base_w = wid * per_w
        lane = lax.iota(jnp.int32, LANES)

        def fire_idx(c, buf):
            base = base_w + c * GATHER_CHUNK
            for j in range(GATHER_CHUNK // LANES):
                q = q0 + base + j * LANES + lane
                t = lax.shift_right_logical(q, 10)
                b = lax.bitwise_and(q, B - 1)
                pos_v[buf, pl.ds(j * LANES, LANES)] = b * T + t
            pltpu.async_copy(idx_hbm.at[pos_v.at[buf]], idxg_v.at[buf],
                             sem_i.at[buf])

        def wait_idx(buf):
            pltpu.make_async_copy(idx_hbm.at[pl.ds(0, GATHER_CHUNK)],
                                  idxg_v.at[buf], sem_i.at[buf]).wait()

        def fire_rows(buf):
            pltpu.async_copy(table_hbm.at[idxg_v.at[buf]], rows_v.at[buf],
                             sem_r.at[buf])

        def wait_rows(buf):
            pltpu.make_async_copy(table_hbm.at[pl.ds(0, GATHER_CHUNK)],
                                  rows_v.at[buf], sem_r.at[buf]).wait()

        def write_out(c, buf):
            base = base_w + c * GATHER_CHUNK
            pltpu.sync_copy(rows_v.at[buf],
                            out_hbm.at[pl.ds(base, GATHER_CHUNK)])

        fire_idx(0, 0)
        wait_idx(0)
        fire_rows(0)
        fire_idx(1, 1)

        @pl.loop(0, chunks, step=2)
        def _chunk(c):
            wait_idx(1)
            fire_rows(1)
            wait_rows(0)
            write_out(c, 0)

            @pl.when(c + 2 < chunks)
            def _next_even():
                fire_idx(c + 2, 0)
                wait_idx(0)
                fire_rows(0)

            wait_rows(1)
            write_out(c + 1, 1)

            @pl.when(c + 3 < chunks)
            def _next_odd():
                fire_idx(c + 3, 1)

    return gather_kernel(table, idx_flat)


def _lstm_body_first(xs_ref, wi_ref, wh_ref, b_ref, ys_ref, h_ref, c_ref):
    gi = pl.program_id(0)

    @pl.when(gi == 0)
    def _init():
        h_ref[...] = jnp.zeros_like(h_ref)
        c_ref[...] = jnp.zeros_like(c_ref)

    _lstm_steps(xs_ref, wi_ref, wh_ref, b_ref, ys_ref, h_ref, c_ref)


def _lstm_body_cont(ys_in_ref, xs_ref, wi_ref, wh_ref, b_ref, h0_ref, c0_ref,
                    ys_ref, h_ref, c_ref):
    del ys_in_ref
    gi = pl.program_id(0)

    @pl.when(gi == 0)
    def _init():
        h_ref[...] = h0_ref[...]
        c_ref[...] = c0_ref[...]

    _lstm_steps(xs_ref, wi_ref, wh_ref, b_ref, ys_ref, h_ref, c_ref)


def _lstm_steps(xs_ref, wi_ref, wh_ref, b_ref, ys_ref, h_ref, c_ref):
    h = h_ref[...]
    c = c_ref[...]
    wi = wi_ref[...]
    wh = wh_ref[...]
    bias = b_ref[...]
    for k in range(S_PER_BLOCK):
        x = xs_ref[k * B:(k + 1) * B, 0:EMBED]
        gates = (
            lax.dot_general(wi, x, (((1,), (1,)), ((), ())),
                            preferred_element_type=jnp.float32)
            + lax.dot_general(wh, h, (((1,), (0,)), ((), ())),
                              preferred_element_type=jnp.float32)
            + bias
        )
        gi_ = jax.nn.sigmoid(gates[0 * HIDDEN:1 * HIDDEN, :])
        gf = jax.nn.sigmoid(gates[1 * HIDDEN:2 * HIDDEN, :])
        gg = jnp.tanh(gates[2 * HIDDEN:3 * HIDDEN, :])
        go = jax.nn.sigmoid(gates[3 * HIDDEN:4 * HIDDEN, :])
        c = gf * c + gi_ * gg
        h = go * jnp.tanh(c)
        ys_ref[k * HIDDEN:(k + 1) * HIDDEN, :] = h
    h_ref[...] = h
    c_ref[...] = c


_WEIGHT_SPECS = [
    pl.BlockSpec((4 * HIDDEN, EMBED), lambda i: (0, 0)),
    pl.BlockSpec((4 * HIDDEN, HIDDEN), lambda i: (0, 0)),
    pl.BlockSpec((4 * HIDDEN, 1), lambda i: (0, 0)),
]
_HC_SPEC = pl.BlockSpec((HIDDEN, B), lambda i: (0, 0))


def _tc_lstm_first(xs1, wi, wh, bias):
    nblk = T1 // S_PER_BLOCK
    return pl.pallas_call(
        _lstm_body_first,
        grid=(nblk,),
        in_specs=[
            pl.BlockSpec((S_PER_BLOCK * B, 2 * EMBED), lambda i: (i, 0)),
            *_WEIGHT_SPECS,
        ],
        out_specs=[
            pl.BlockSpec((S_PER_BLOCK * HIDDEN, B), lambda i: (i, 0)),
            _HC_SPEC,
            _HC_SPEC,
        ],
        out_shape=[
            jax.ShapeDtypeStruct((T * HIDDEN, B), jnp.float32),
            jax.ShapeDtypeStruct((HIDDEN, B), jnp.float32),
            jax.ShapeDtypeStruct((HIDDEN, B), jnp.float32),
        ],
        compiler_params=pltpu.CompilerParams(
            dimension_semantics=("arbitrary",),
        ),
    )(xs1, wi, wh, bias)


def _tc_lstm_cont(ys_prev, xs2, wi, wh, bias, h0, c0):
    nblk = T2 // S_PER_BLOCK
    blk0 = T1 // S_PER_BLOCK
    return pl.pallas_call(
        _lstm_body_cont,
        grid=(nblk,),
        in_specs=[
            pl.BlockSpec(memory_space=pl.ANY),
            pl.BlockSpec((S_PER_BLOCK * B, 2 * EMBED), lambda i: (i, 0)),
            *_WEIGHT_SPECS,
            _HC_SPEC,
            _HC_SPEC,
        ],
        out_specs=[
            pl.BlockSpec((S_PER_BLOCK * HIDDEN, B),
                         lambda i: (i + blk0, 0)),
            _HC_SPEC,
            _HC_SPEC,
        ],
        out_shape=[
            jax.ShapeDtypeStruct((T * HIDDEN, B), jnp.float32),
            jax.ShapeDtypeStruct((HIDDEN, B), jnp.float32),
            jax.ShapeDtypeStruct((HIDDEN, B), jnp.float32),
        ],
        input_output_aliases={0: 0},
        compiler_params=pltpu.CompilerParams(
            dimension_semantics=("arbitrary",),
        ),
    )(ys_prev, xs2, wi, wh, bias, h0, c0)


def kernel(src, table, W_ih, W_hh, b_ih, b_hh):
    idx_flat = src.reshape(-1)
    t128 = jnp.pad(table, ((0, 0), (0, EMBED)))
    xs1 = _sc_gather(t128, idx_flat, 0, T1 * B)
    xs2 = _sc_gather(t128, idx_flat, T1 * B, T2 * B)

    bias = (b_ih + b_hh)[:, None]

    ys1, h1, c1 = _tc_lstm_first(xs1, W_ih, W_hh, bias)
    ysT, hT, cT = _tc_lstm_cont(ys1, xs2, W_ih, W_hh, bias, h1, c1)
    outputs = ysT.reshape(T, HIDDEN, B).transpose(2, 0, 1)
    hidden = hT.transpose(1, 0)[None]
    cell = cT.transpose(1, 0)[None]
    return (outputs, hidden, cell)


# speedup vs baseline: 1.6132x; 1.0138x over previous
import functools

import jax
import jax.numpy as jnp
from jax import lax
from jax.experimental import pallas as pl
from jax.experimental.pallas import tpu as pltpu
from jax.experimental.pallas import tpu_sc as plsc

VOCAB = 1000000
EMBED = 64
HIDDEN = 64
B = 1024
T = 200
T_PARTS = (64, 64, 72)

NUM_CORES = 2
NUM_SUBCORES = 16
NUM_WORKERS = NUM_CORES * NUM_SUBCORES
LANES = 16

GATHER_CHUNK = 128

S_PER_BLOCK = 8


def _sc_gather(table, idx_flat, q0, n):
    per_w = n // NUM_WORKERS
    chunks = per_w // GATHER_CHUNK
    assert per_w * NUM_WORKERS == n and chunks * GATHER_CHUNK == per_w
    assert chunks % 2 == 0 and q0 % GATHER_CHUNK == 0

    mesh = plsc.VectorSubcoreMesh(core_axis_name="c", subcore_axis_name="s")

    @functools.partial(
        pl.kernel,
        out_type=jax.ShapeDtypeStruct((n, 2 * EMBED), jnp.float32),
        mesh=mesh,
        scratch_types=[
            pltpu.VMEM((2, GATHER_CHUNK), jnp.int32),
            pltpu.VMEM((2, GATHER_CHUNK), jnp.int32),
            pltpu.VMEM((2, GATHER_CHUNK, 2 * EMBED), jnp.float32),
            pltpu.SemaphoreType.DMA((2,)),
            pltpu.SemaphoreType.DMA((2,)),
        ],
        compiler_params=pltpu.CompilerParams(use_tc_tiling_on_sc=False),
    )
    def gather_kernel(table_hbm, idx_hbm, out_hbm, pos_v, idxg_v, rows_v,
                      sem_i, sem_r):
        wid = lax.axis_index("s") * NUM_CORES + lax.axis_index("c")
        base_w = wid * per_w
        lane = lax.iota(jnp.int32, LANES)

        def fire_idx(c, buf):
            base = base_w + c * GATHER_CHUNK
            for j in range(GATHER_CHUNK // LANES):
                q = q0 + base + j * LANES + lane
                t = lax.shift_right_logical(q, 10)
                b = lax.bitwise_and(q, B - 1)
                pos_v[buf, pl.ds(j * LANES, LANES)] = b * T + t
            pltpu.async_copy(idx_hbm.at[pos_v.at[buf]], idxg_v.at[buf],
                             sem_i.at[buf])

        def wait_idx(buf):
            pltpu.make_async_copy(idx_hbm.at[pl.ds(0, GATHER_CHUNK)],
                                  idxg_v.at[buf], sem_i.at[buf]).wait()

        def fire_rows(buf):
            pltpu.async_copy(table_hbm.at[idxg_v.at[buf]], rows_v.at[buf],
                             sem_r.at[buf])

        def wait_rows(buf):
            pltpu.make_async_copy(table_hbm.at[pl.ds(0, GATHER_CHUNK)],
                                  rows_v.at[buf], sem_r.at[buf]).wait()

        def write_out(c, buf):
            base = base_w + c * GATHER_CHUNK
            pltpu.sync_copy(rows_v.at[buf],
                            out_hbm.at[pl.ds(base, GATHER_CHUNK)])

        fire_idx(0, 0)
        wait_idx(0)
        fire_rows(0)
        fire_idx(1, 1)

        @pl.loop(0, chunks, step=2)
        def _chunk(c):
            wait_idx(1)
            fire_rows(1)
            wait_rows(0)
            write_out(c, 0)

            @pl.when(c + 2 < chunks)
            def _next_even():
                fire_idx(c + 2, 0)
                wait_idx(0)
                fire_rows(0)

            wait_rows(1)
            write_out(c + 1, 1)

            @pl.when(c + 3 < chunks)
            def _next_odd():
                fire_idx(c + 3, 1)

    return gather_kernel(table, idx_flat)


def _lstm_body_first(xs_ref, wi_ref, wh_ref, b_ref, ys_ref, h_ref, c_ref):
    gi = pl.program_id(0)

    @pl.when(gi == 0)
    def _init():
        h_ref[...] = jnp.zeros_like(h_ref)
        c_ref[...] = jnp.zeros_like(c_ref)

    _lstm_steps(xs_ref, wi_ref, wh_ref, b_ref, ys_ref, h_ref, c_ref)


def _lstm_body_cont(ys_in_ref, xs_ref, wi_ref, wh_ref, b_ref, h0_ref, c0_ref,
                    ys_ref, h_ref, c_ref):
    del ys_in_ref
    gi = pl.program_id(0)

    @pl.when(gi == 0)
    def _init():
        h_ref[...] = h0_ref[...]
        c_ref[...] = c0_ref[...]

    _lstm_steps(xs_ref, wi_ref, wh_ref, b_ref, ys_ref, h_ref, c_ref)


def _lstm_steps(xs_ref, wi_ref, wh_ref, b_ref, ys_ref, h_ref, c_ref):
    h = h_ref[...]
    c = c_ref[...]
    wi = wi_ref[...]
    wh = wh_ref[...]
    bias = b_ref[...]
    for k in range(S_PER_BLOCK):
        x = xs_ref[k * B:(k + 1) * B, 0:EMBED]
        gates = (
            lax.dot_general(wi, x, (((1,), (1,)), ((), ())),
                            preferred_element_type=jnp.float32)
            + lax.dot_general(wh, h, (((1,), (0,)), ((), ())),
                              preferred_element_type=jnp.float32)
            + bias
        )
        gi_ = jax.nn.sigmoid(gates[0 * HIDDEN:1 * HIDDEN, :])
        gf = jax.nn.sigmoid(gates[1 * HIDDEN:2 * HIDDEN, :])
        gg = jnp.tanh(gates[2 * HIDDEN:3 * HIDDEN, :])
        go = jax.nn.sigmoid(gates[3 * HIDDEN:4 * HIDDEN, :])
        c = gf * c + gi_ * gg
        h = go * jnp.tanh(c)
        ys_ref[k * HIDDEN:(k + 1) * HIDDEN, :] = h
    h_ref[...] = h
    c_ref[...] = c


_WEIGHT_SPECS = [
    pl.BlockSpec((4 * HIDDEN, EMBED), lambda i: (0, 0)),
    pl.BlockSpec((4 * HIDDEN, HIDDEN), lambda i: (0, 0)),
    pl.BlockSpec((4 * HIDDEN, 1), lambda i: (0, 0)),
]
_HC_SPEC = pl.BlockSpec((HIDDEN, B), lambda i: (0, 0))


def _tc_lstm_first(xs1, wi, wh, bias, t_part):
    nblk = t_part // S_PER_BLOCK
    return pl.pallas_call(
        _lstm_body_first,
        grid=(nblk,),
        in_specs=[
            pl.BlockSpec((S_PER_BLOCK * B, 2 * EMBED), lambda i: (i, 0)),
            *_WEIGHT_SPECS,
        ],
        out_specs=[
            pl.BlockSpec((S_PER_BLOCK * HIDDEN, B), lambda i: (i, 0)),
            _HC_SPEC,
            _HC_SPEC,
        ],
        out_shape=[
            jax.ShapeDtypeStruct((T * HIDDEN, B), jnp.float32),
            jax.ShapeDtypeStruct((HIDDEN, B), jnp.float32),
            jax.ShapeDtypeStruct((HIDDEN, B), jnp.float32),
        ],
        compiler_params=pltpu.CompilerParams(
            dimension_semantics=("arbitrary",),
        ),
    )(xs1, wi, wh, bias)


def _tc_lstm_cont(ys_prev, xs2, wi, wh, bias, h0, c0, t_part, t0):
    nblk = t_part // S_PER_BLOCK
    blk0 = t0 // S_PER_BLOCK
    return pl.pallas_call(
        _lstm_body_cont,
        grid=(nblk,),
        in_specs=[
            pl.BlockSpec(memory_space=pl.ANY),
            pl.BlockSpec((S_PER_BLOCK * B, 2 * EMBED), lambda i: (i, 0)),
            *_WEIGHT_SPECS,
            _HC_SPEC,
            _HC_SPEC,
        ],
        out_specs=[
            pl.BlockSpec((S_PER_BLOCK * HIDDEN, B),
                         lambda i: (i + blk0, 0)),
            _HC_SPEC,
            _HC_SPEC,
        ],
        out_shape=[
            jax.ShapeDtypeStruct((T * HIDDEN, B), jnp.float32),
            jax.ShapeDtypeStruct((HIDDEN, B), jnp.float32),
            jax.ShapeDtypeStruct((HIDDEN, B), jnp.float32),
        ],
        input_output_aliases={0: 0},
        compiler_params=pltpu.CompilerParams(
            dimension_semantics=("arbitrary",),
        ),
    )(ys_prev, xs2, wi, wh, bias, h0, c0)


def kernel(src, table, W_ih, W_hh, b_ih, b_hh):
    idx_flat = src.reshape(-1)
    t128 = jnp.pad(table, ((0, 0), (0, EMBED)))

    starts = [sum(T_PARTS[:i]) for i in range(len(T_PARTS))]
    xs_parts = [
        _sc_gather(t128, idx_flat, t0 * B, tp * B)
        for t0, tp in zip(starts, T_PARTS)
    ]

    bias = (b_ih + b_hh)[:, None]

    ysT, hT, cT = _tc_lstm_first(xs_parts[0], W_ih, W_hh, bias, T_PARTS[0])
    for i in range(1, len(T_PARTS)):
        ysT, hT, cT = _tc_lstm_cont(ysT, xs_parts[i], W_ih, W_hh, bias,
                                    hT, cT, T_PARTS[i], starts[i])
    outputs = ysT.reshape(T, HIDDEN, B).transpose(2, 0, 1)
    hidden = hT.transpose(1, 0)[None]
    cell = cT.transpose(1, 0)[None]
    return (outputs, hidden, cell)


# speedup vs baseline: 2.5201x; 1.5622x over previous
import functools

import jax
import jax.numpy as jnp
from jax import lax
from jax.experimental import pallas as pl
from jax.experimental.pallas import tpu as pltpu
from jax.experimental.pallas import tpu_sc as plsc

VOCAB = 1000000
EMBED = 64
HIDDEN = 64
B = 1024
T = 200
T_PARTS = (64, 64, 72)

NUM_CORES = 2
NUM_SUBCORES = 16
NUM_WORKERS = NUM_CORES * NUM_SUBCORES
LANES = 16

GATHER_CHUNK = 128

S_PER_BLOCK = 8


TROWS = 8192


def _transpose_body(tt_ref, out_ref):
    out_ref[:, 0:EMBED] = jnp.swapaxes(tt_ref[...], 0, 1)


def _tc_transpose_pad(tableT):
    vpad = ((VOCAB + TROWS - 1) // TROWS) * TROWS
    nblk = vpad // TROWS
    return pl.pallas_call(
        _transpose_body,
        grid=(nblk,),
        in_specs=[pl.BlockSpec((EMBED, TROWS), lambda i: (0, i))],
        out_specs=pl.BlockSpec((TROWS, 2 * EMBED), lambda i: (i, 0)),
        out_shape=jax.ShapeDtypeStruct((vpad, 2 * EMBED), jnp.float32),
        compiler_params=pltpu.CompilerParams(
            dimension_semantics=("arbitrary",),
        ),
    )(tableT)


def _sc_gather(table, idx_flat, q0, n):
    per_w = n // NUM_WORKERS
    chunks = per_w // GATHER_CHUNK
    assert per_w * NUM_WORKERS == n and chunks * GATHER_CHUNK == per_w
    assert chunks % 2 == 0 and q0 % GATHER_CHUNK == 0

    mesh = plsc.VectorSubcoreMesh(core_axis_name="c", subcore_axis_name="s")

    @functools.partial(
        pl.kernel,
        out_type=jax.ShapeDtypeStruct((n, 2 * EMBED), jnp.float32),
        mesh=mesh,
        scratch_types=[
            pltpu.VMEM((2, GATHER_CHUNK), jnp.int32),
            pltpu.VMEM((2, GATHER_CHUNK), jnp.int32),
            pltpu.VMEM((2, GATHER_CHUNK, 2 * EMBED), jnp.float32),
            pltpu.SemaphoreType.DMA((2,)),
            pltpu.SemaphoreType.DMA((2,)),
        ],
        compiler_params=pltpu.CompilerParams(use_tc_tiling_on_sc=False),
    )
    def gather_kernel(table_hbm, idx_hbm, out_hbm, pos_v, idxg_v, rows_v,
                      sem_i, sem_r):
        wid = lax.axis_index("s") * NUM_CORES + lax.axis_index("c")
        base_w = wid * per_w
        lane = lax.iota(jnp.int32, LANES)

        def fire_idx(c, buf):
            base = base_w + c * GATHER_CHUNK
            for j in range(GATHER_CHUNK // LANES):
                q = q0 + base + j * LANES + lane
                t = lax.shift_right_logical(q, 10)
                b = lax.bitwise_and(q, B - 1)
                pos_v[buf, pl.ds(j * LANES, LANES)] = b * T + t
            pltpu.async_copy(idx_hbm.at[pos_v.at[buf]], idxg_v.at[buf],
                             sem_i.at[buf])

        def wait_idx(buf):
            pltpu.make_async_copy(idx_hbm.at[pl.ds(0, GATHER_CHUNK)],
                                  idxg_v.at[buf], sem_i.at[buf]).wait()

        def fire_rows(buf):
            pltpu.async_copy(table_hbm.at[idxg_v.at[buf]], rows_v.at[buf],
                             sem_r.at[buf])

        def wait_rows(buf):
            pltpu.make_async_copy(table_hbm.at[pl.ds(0, GATHER_CHUNK)],
                                  rows_v.at[buf], sem_r.at[buf]).wait()

        def write_out(c, buf):
            base = base_w + c * GATHER_CHUNK
            pltpu.sync_copy(rows_v.at[buf],
                            out_hbm.at[pl.ds(base, GATHER_CHUNK)])

        fire_idx(0, 0)
        wait_idx(0)
        fire_rows(0)
        fire_idx(1, 1)

        @pl.loop(0, chunks, step=2)
        def _chunk(c):
            wait_idx(1)
            fire_rows(1)
            wait_rows(0)
            write_out(c, 0)

            @pl.when(c + 2 < chunks)
            def _next_even():
                fire_idx(c + 2, 0)
                wait_idx(0)
                fire_rows(0)

            wait_rows(1)
            write_out(c + 1, 1)

            @pl.when(c + 3 < chunks)
            def _next_odd():
                fire_idx(c + 3, 1)

    return gather_kernel(table, idx_flat)


def _lstm_body_first(xs_ref, wi_ref, wh_ref, b_ref, ys_ref, h_ref, c_ref):
    gi = pl.program_id(0)

    @pl.when(gi == 0)
    def _init():
        h_ref[...] = jnp.zeros_like(h_ref)
        c_ref[...] = jnp.zeros_like(c_ref)

    _lstm_steps(xs_ref, wi_ref, wh_ref, b_ref, ys_ref, h_ref, c_ref)


def _lstm_body_cont(ys_in_ref, xs_ref, wi_ref, wh_ref, b_ref, h0_ref, c0_ref,
                    ys_ref, h_ref, c_ref):
    del ys_in_ref
    gi = pl.program_id(0)

    @pl.when(gi == 0)
    def _init():
        h_ref[...] = h0_ref[...]
        c_ref[...] = c0_ref[...]

    _lstm_steps(xs_ref, wi_ref, wh_ref, b_ref, ys_ref, h_ref, c_ref)


def _lstm_steps(xs_ref, wi_ref, wh_ref, b_ref, ys_ref, h_ref, c_ref):
    h = h_ref[...]
    c = c_ref[...]
    wi = wi_ref[...]
    wh = wh_ref[...]
    bias = b_ref[...]
    for k in range(S_PER_BLOCK):
        x = xs_ref[k * B:(k + 1) * B, 0:EMBED]
        gates = (
            lax.dot_general(wi, x, (((1,), (1,)), ((), ())),
                            preferred_element_type=jnp.float32)
            + lax.dot_general(wh, h, (((1,), (0,)), ((), ())),
                              preferred_element_type=jnp.float32)
            + bias
        )
        gi_ = jax.nn.sigmoid(gates[0 * HIDDEN:1 * HIDDEN, :])
        gf = jax.nn.sigmoid(gates[1 * HIDDEN:2 * HIDDEN, :])
        gg = jnp.tanh(gates[2 * HIDDEN:3 * HIDDEN, :])
        go = jax.nn.sigmoid(gates[3 * HIDDEN:4 * HIDDEN, :])
        c = gf * c + gi_ * gg
        h = go * jnp.tanh(c)
        ys_ref[k * HIDDEN:(k + 1) * HIDDEN, :] = h
    h_ref[...] = h
    c_ref[...] = c


_WEIGHT_SPECS = [
    pl.BlockSpec((4 * HIDDEN, EMBED), lambda i: (0, 0)),
    pl.BlockSpec((4 * HIDDEN, HIDDEN), lambda i: (0, 0)),
    pl.BlockSpec((4 * HIDDEN, 1), lambda i: (0, 0)),
]
_HC_SPEC = pl.BlockSpec((HIDDEN, B), lambda i: (0, 0))


def _tc_lstm_first(xs1, wi, wh, bias, t_part):
    nblk = t_part // S_PER_BLOCK
    return pl.pallas_call(
        _lstm_body_first,
        grid=(nblk,),
        in_specs=[
            pl.BlockSpec((S_PER_BLOCK * B, 2 * EMBED), lambda i: (i, 0)),
            *_WEIGHT_SPECS,
        ],
        out_specs=[
            pl.BlockSpec((S_PER_BLOCK * HIDDEN, B), lambda i: (i, 0)),
            _HC_SPEC,
            _HC_SPEC,
        ],
        out_shape=[
            jax.ShapeDtypeStruct((T * HIDDEN, B), jnp.float32),
            jax.ShapeDtypeStruct((HIDDEN, B), jnp.float32),
            jax.ShapeDtypeStruct((HIDDEN, B), jnp.float32),
        ],
        compiler_params=pltpu.CompilerParams(
            dimension_semantics=("arbitrary",),
        ),
    )(xs1, wi, wh, bias)


def _tc_lstm_cont(ys_prev, xs2, wi, wh, bias, h0, c0, t_part, t0):
    nblk = t_part // S_PER_BLOCK
    blk0 = t0 // S_PER_BLOCK
    return pl.pallas_call(
        _lstm_body_cont,
        grid=(nblk,),
        in_specs=[
            pl.BlockSpec(memory_space=pl.ANY),
            pl.BlockSpec((S_PER_BLOCK * B, 2 * EMBED), lambda i: (i, 0)),
            *_WEIGHT_SPECS,
            _HC_SPEC,
            _HC_SPEC,
        ],
        out_specs=[
            pl.BlockSpec((S_PER_BLOCK * HIDDEN, B),
                         lambda i: (i + blk0, 0)),
            _HC_SPEC,
            _HC_SPEC,
        ],
        out_shape=[
            jax.ShapeDtypeStruct((T * HIDDEN, B), jnp.float32),
            jax.ShapeDtypeStruct((HIDDEN, B), jnp.float32),
            jax.ShapeDtypeStruct((HIDDEN, B), jnp.float32),
        ],
        input_output_aliases={0: 0},
        compiler_params=pltpu.CompilerParams(
            dimension_semantics=("arbitrary",),
        ),
    )(ys_prev, xs2, wi, wh, bias, h0, c0)


def kernel(src, table, W_ih, W_hh, b_ih, b_hh):
    idx_flat = src.reshape(-1)
    t128 = _tc_transpose_pad(jnp.transpose(table))

    starts = [sum(T_PARTS[:i]) for i in range(len(T_PARTS))]
    xs_parts = [
        _sc_gather(t128, idx_flat, t0 * B, tp * B)
        for t0, tp in zip(starts, T_PARTS)
    ]

    bias = (b_ih + b_hh)[:, None]

    ysT, hT, cT = _tc_lstm_first(xs_parts[0], W_ih, W_hh, bias, T_PARTS[0])
    for i in range(1, len(T_PARTS)):
        ysT, hT, cT = _tc_lstm_cont(ysT, xs_parts[i], W_ih, W_hh, bias,
                                    hT, cT, T_PARTS[i], starts[i])
    outputs = ysT.reshape(T, HIDDEN, B).transpose(2, 0, 1)
    hidden = hT.transpose(1, 0)[None]
    cell = cT.transpose(1, 0)[None]
    return (outputs, hidden, cell)


# speedup vs baseline: 2.6288x; 1.0431x over previous
import functools

import jax
import jax.numpy as jnp
from jax import lax
from jax.experimental import pallas as pl
from jax.experimental.pallas import tpu as pltpu
from jax.experimental.pallas import tpu_sc as plsc

VOCAB = 1000000
EMBED = 64
HIDDEN = 64
B = 1024
T = 200
T_PARTS = (64, 64, 72)

NUM_CORES = 2
NUM_SUBCORES = 16
NUM_WORKERS = NUM_CORES * NUM_SUBCORES
LANES = 16

GATHER_CHUNK = 128

S_PER_BLOCK = 8


TROWS = 16384


def _transpose_body(tt_ref, out_ref):
    out_ref[:, 0:EMBED] = jnp.swapaxes(tt_ref[...], 0, 1)


def _tc_transpose_pad(tableT):
    vpad = ((VOCAB + TROWS - 1) // TROWS) * TROWS
    nblk = vpad // TROWS
    return pl.pallas_call(
        _transpose_body,
        grid=(nblk,),
        in_specs=[pl.BlockSpec((EMBED, TROWS), lambda i: (0, i))],
        out_specs=pl.BlockSpec((TROWS, 2 * EMBED), lambda i: (i, 0)),
        out_shape=jax.ShapeDtypeStruct((vpad, 2 * EMBED), jnp.float32),
        compiler_params=pltpu.CompilerParams(
            dimension_semantics=("arbitrary",),
        ),
    )(tableT)


def _sc_gather(table, idx_flat, q0, n):
    per_w = n // NUM_WORKERS
    chunks = per_w // GATHER_CHUNK
    assert per_w * NUM_WORKERS == n and chunks * GATHER_CHUNK == per_w
    assert chunks % 2 == 0 and q0 % GATHER_CHUNK == 0

    mesh = plsc.VectorSubcoreMesh(core_axis_name="c", subcore_axis_name="s")

    @functools.partial(
        pl.kernel,
        out_type=jax.ShapeDtypeStruct((n, 2 * EMBED), jnp.float32),
        mesh=mesh,
        scratch_types=[
            pltpu.VMEM((2, GATHER_CHUNK), jnp.int32),
            pltpu.VMEM((2, GATHER_CHUNK), jnp.int32),
            pltpu.VMEM((2, GATHER_CHUNK, 2 * EMBED), jnp.float32),
            pltpu.SemaphoreType.DMA((2,)),
            pltpu.SemaphoreType.DMA((2,)),
        ],
        compiler_params=pltpu.CompilerParams(use_tc_tiling_on_sc=False),
    )
    def gather_kernel(table_hbm, idx_hbm, out_hbm, pos_v, idxg_v, rows_v,
                      sem_i, sem_r):
        wid = lax.axis_index("s") * NUM_CORES + lax.axis_index("c")
        base_w = wid * per_w
        lane = lax.iota(jnp.int32, LANES)

        def fire_idx(c, buf):
            base = base_w + c * GATHER_CHUNK
            for j in range(GATHER_CHUNK // LANES):
                q = q0 + base + j * LANES + lane
                t = lax.shift_right_logical(q, 10)
                b = lax.bitwise_and(q, B - 1)
                pos_v[buf, pl.ds(j * LANES, LANES)] = b * T + t
            pltpu.async_copy(idx_hbm.at[pos_v.at[buf]], idxg_v.at[buf],
                             sem_i.at[buf])

        def wait_idx(buf):
            pltpu.make_async_copy(idx_hbm.at[pl.ds(0, GATHER_CHUNK)],
                                  idxg_v.at[buf], sem_i.at[buf]).wait()

        def fire_rows(buf):
            pltpu.async_copy(table_hbm.at[idxg_v.at[buf]], rows_v.at[buf],
                             sem_r.at[buf])

        def wait_rows(buf):
            pltpu.make_async_copy(table_hbm.at[pl.ds(0, GATHER_CHUNK)],
                                  rows_v.at[buf], sem_r.at[buf]).wait()

        def write_out(c, buf):
            base = base_w + c * GATHER_CHUNK
            pltpu.sync_copy(rows_v.at[buf],
                            out_hbm.at[pl.ds(base, GATHER_CHUNK)])

        fire_idx(0, 0)
        wait_idx(0)
        fire_rows(0)
        fire_idx(1, 1)

        @pl.loop(0, chunks, step=2)
        def _chunk(c):
            wait_idx(1)
            fire_rows(1)
            wait_rows(0)
            write_out(c, 0)

            @pl.when(c + 2 < chunks)
            def _next_even():
                fire_idx(c + 2, 0)
                wait_idx(0)
                fire_rows(0)

            wait_rows(1)
            write_out(c + 1, 1)

            @pl.when(c + 3 < chunks)
            def _next_odd():
                fire_idx(c + 3, 1)

    return gather_kernel(table, idx_flat)


def _lstm_body_first(xs_ref, wi_ref, wh_ref, b_ref, ys_ref, h_ref, c_ref):
    gi = pl.program_id(0)

    @pl.when(gi == 0)
    def _init():
        h_ref[...] = jnp.zeros_like(h_ref)
        c_ref[...] = jnp.zeros_like(c_ref)

    _lstm_steps(xs_ref, wi_ref, wh_ref, b_ref, ys_ref, h_ref, c_ref)


def _lstm_body_cont(ys_in_ref, xs_ref, wi_ref, wh_ref, b_ref, h0_ref, c0_ref,
                    ys_ref, h_ref, c_ref):
    del ys_in_ref
    gi = pl.program_id(0)

    @pl.when(gi == 0)
    def _init():
        h_ref[...] = h0_ref[...]
        c_ref[...] = c0_ref[...]

    _lstm_steps(xs_ref, wi_ref, wh_ref, b_ref, ys_ref, h_ref, c_ref)


def _lstm_steps(xs_ref, wi_ref, wh_ref, b_ref, ys_ref, h_ref, c_ref):
    h = h_ref[...]
    c = c_ref[...]
    wi = wi_ref[...]
    wh = wh_ref[...]
    bias = b_ref[...]
    for k in range(S_PER_BLOCK):
        x = xs_ref[k * B:(k + 1) * B, 0:EMBED]
        gates = (
            lax.dot_general(wi, x, (((1,), (1,)), ((), ())),
                            preferred_element_type=jnp.float32)
            + lax.dot_general(wh, h, (((1,), (0,)), ((), ())),
                              preferred_element_type=jnp.float32)
            + bias
        )
        gi_ = jax.nn.sigmoid(gates[0 * HIDDEN:1 * HIDDEN, :])
        gf = jax.nn.sigmoid(gates[1 * HIDDEN:2 * HIDDEN, :])
        gg = jnp.tanh(gates[2 * HIDDEN:3 * HIDDEN, :])
        go = jax.nn.sigmoid(gates[3 * HIDDEN:4 * HIDDEN, :])
        c = gf * c + gi_ * gg
        h = go * jnp.tanh(c)
        ys_ref[k * HIDDEN:(k + 1) * HIDDEN, :] = h
    h_ref[...] = h
    c_ref[...] = c


_WEIGHT_SPECS = [
    pl.BlockSpec((4 * HIDDEN, EMBED), lambda i: (0, 0)),
    pl.BlockSpec((4 * HIDDEN, HIDDEN), lambda i: (0, 0)),
    pl.BlockSpec((4 * HIDDEN, 1), lambda i: (0, 0)),
]
_HC_SPEC = pl.BlockSpec((HIDDEN, B), lambda i: (0, 0))


def _tc_lstm_first(xs1, wi, wh, bias, t_part):
    nblk = t_part // S_PER_BLOCK
    return pl.pallas_call(
        _lstm_body_first,
        grid=(nblk,),
        in_specs=[
            pl.BlockSpec((S_PER_BLOCK * B, 2 * EMBED), lambda i: (i, 0)),
            *_WEIGHT_SPECS,
        ],
        out_specs=[
            pl.BlockSpec((S_PER_BLOCK * HIDDEN, B), lambda i: (i, 0)),
            _HC_SPEC,
            _HC_SPEC,
        ],
        out_shape=[
            jax.ShapeDtypeStruct((T * HIDDEN, B), jnp.float32),
            jax.ShapeDtypeStruct((HIDDEN, B), jnp.float32),
            jax.ShapeDtypeStruct((HIDDEN, B), jnp.float32),
        ],
        compiler_params=pltpu.CompilerParams(
            dimension_semantics=("arbitrary",),
        ),
    )(xs1, wi, wh, bias)


def _tc_lstm_cont(ys_prev, xs2, wi, wh, bias, h0, c0, t_part, t0):
    nblk = t_part // S_PER_BLOCK
    blk0 = t0 // S_PER_BLOCK
    return pl.pallas_call(
        _lstm_body_cont,
        grid=(nblk,),
        in_specs=[
            pl.BlockSpec(memory_space=pl.ANY),
            pl.BlockSpec((S_PER_BLOCK * B, 2 * EMBED), lambda i: (i, 0)),
            *_WEIGHT_SPECS,
            _HC_SPEC,
            _HC_SPEC,
        ],
        out_specs=[
            pl.BlockSpec((S_PER_BLOCK * HIDDEN, B),
                         lambda i: (i + blk0, 0)),
            _HC_SPEC,
            _HC_SPEC,
        ],
        out_shape=[
            jax.ShapeDtypeStruct((T * HIDDEN, B), jnp.float32),
            jax.ShapeDtypeStruct((HIDDEN, B), jnp.float32),
            jax.ShapeDtypeStruct((HIDDEN, B), jnp.float32),
        ],
        input_output_aliases={0: 0},
        compiler_params=pltpu.CompilerParams(
            dimension_semantics=("arbitrary",),
        ),
    )(ys_prev, xs2, wi, wh, bias, h0, c0)


def kernel(src, table, W_ih, W_hh, b_ih, b_hh):
    idx_flat = src.reshape(-1)
    t128 = _tc_transpose_pad(jnp.transpose(table))

    starts = [sum(T_PARTS[:i]) for i in range(len(T_PARTS))]
    xs_parts = [
        _sc_gather(t128, idx_flat, t0 * B, tp * B)
        for t0, tp in zip(starts, T_PARTS)
    ]

    bias = (b_ih + b_hh)[:, None]

    ysT, hT, cT = _tc_lstm_first(xs_parts[0], W_ih, W_hh, bias, T_PARTS[0])
    for i in range(1, len(T_PARTS)):
        ysT, hT, cT = _tc_lstm_cont(ysT, xs_parts[i], W_ih, W_hh, bias,
                                    hT, cT, T_PARTS[i], starts[i])
    outputs = ysT.reshape(T, HIDDEN, B).transpose(2, 0, 1)
    hidden = hT.transpose(1, 0)[None]
    cell = cT.transpose(1, 0)[None]
    return (outputs, hidden, cell)


# speedup vs baseline: 2.6677x; 1.0148x over previous
import functools

import jax
import jax.numpy as jnp
from jax import lax
from jax.experimental import pallas as pl
from jax.experimental.pallas import tpu as pltpu
from jax.experimental.pallas import tpu_sc as plsc

VOCAB = 1000000
EMBED = 64
HIDDEN = 64
B = 1024
T = 200
T_PARTS = (64, 64, 72)

NUM_CORES = 2
NUM_SUBCORES = 16
NUM_WORKERS = NUM_CORES * NUM_SUBCORES
LANES = 16

GATHER_CHUNK = 128

S_PER_BLOCK = 8


TROWS = 32768


def _transpose_body(tt_ref, out_ref):
    out_ref[:, 0:EMBED] = jnp.swapaxes(tt_ref[...], 0, 1)


def _tc_transpose_pad(tableT):
    vpad = ((VOCAB + TROWS - 1) // TROWS) * TROWS
    nblk = vpad // TROWS
    return pl.pallas_call(
        _transpose_body,
        grid=(nblk,),
        in_specs=[pl.BlockSpec((EMBED, TROWS), lambda i: (0, i))],
        out_specs=pl.BlockSpec((TROWS, 2 * EMBED), lambda i: (i, 0)),
        out_shape=jax.ShapeDtypeStruct((vpad, 2 * EMBED), jnp.float32),
        compiler_params=pltpu.CompilerParams(
            dimension_semantics=("arbitrary",),
        ),
    )(tableT)


def _sc_gather(table, idx_flat, q0, n):
    per_w = n // NUM_WORKERS
    chunks = per_w // GATHER_CHUNK
    assert per_w * NUM_WORKERS == n and chunks * GATHER_CHUNK == per_w
    assert chunks % 2 == 0 and q0 % GATHER_CHUNK == 0

    mesh = plsc.VectorSubcoreMesh(core_axis_name="c", subcore_axis_name="s")

    @functools.partial(
        pl.kernel,
        out_type=jax.ShapeDtypeStruct((n, 2 * EMBED), jnp.float32),
        mesh=mesh,
        scratch_types=[
            pltpu.VMEM((2, GATHER_CHUNK), jnp.int32),
            pltpu.VMEM((2, GATHER_CHUNK), jnp.int32),
            pltpu.VMEM((2, GATHER_CHUNK, 2 * EMBED), jnp.float32),
            pltpu.SemaphoreType.DMA((2,)),
            pltpu.SemaphoreType.DMA((2,)),
        ],
        compiler_params=pltpu.CompilerParams(use_tc_tiling_on_sc=False),
    )
    def gather_kernel(table_hbm, idx_hbm, out_hbm, pos_v, idxg_v, rows_v,
                      sem_i, sem_r):
        wid = lax.axis_index("s") * NUM_CORES + lax.axis_index("c")
        base_w = wid * per_w
        lane = lax.iota(jnp.int32, LANES)

        def fire_idx(c, buf):
            base = base_w + c * GATHER_CHUNK
            for j in range(GATHER_CHUNK // LANES):
                q = q0 + base + j * LANES + lane
                t = lax.shift_right_logical(q, 10)
                b = lax.bitwise_and(q, B - 1)
                pos_v[buf, pl.ds(j * LANES, LANES)] = b * T + t
            pltpu.async_copy(idx_hbm.at[pos_v.at[buf]], idxg_v.at[buf],
                             sem_i.at[buf])

        def wait_idx(buf):
            pltpu.make_async_copy(idx_hbm.at[pl.ds(0, GATHER_CHUNK)],
                                  idxg_v.at[buf], sem_i.at[buf]).wait()

        def fire_rows(buf):
            pltpu.async_copy(table_hbm.at[idxg_v.at[buf]], rows_v.at[buf],
                             sem_r.at[buf])

        def wait_rows(buf):
            pltpu.make_async_copy(table_hbm.at[pl.ds(0, GATHER_CHUNK)],
                                  rows_v.at[buf], sem_r.at[buf]).wait()

        def write_out(c, buf):
            base = base_w + c * GATHER_CHUNK
            pltpu.sync_copy(rows_v.at[buf],
                            out_hbm.at[pl.ds(base, GATHER_CHUNK)])

        fire_idx(0, 0)
        wait_idx(0)
        fire_rows(0)
        fire_idx(1, 1)

        @pl.loop(0, chunks, step=2)
        def _chunk(c):
            wait_idx(1)
            fire_rows(1)
            wait_rows(0)
            write_out(c, 0)

            @pl.when(c + 2 < chunks)
            def _next_even():
                fire_idx(c + 2, 0)
                wait_idx(0)
                fire_rows(0)

            wait_rows(1)
            write_out(c + 1, 1)

            @pl.when(c + 3 < chunks)
            def _next_odd():
                fire_idx(c + 3, 1)

    return gather_kernel(table, idx_flat)


def _lstm_body_first(xs_ref, wi_ref, wh_ref, b_ref, ys_ref, h_ref, c_ref):
    gi = pl.program_id(0)

    @pl.when(gi == 0)
    def _init():
        h_ref[...] = jnp.zeros_like(h_ref)
        c_ref[...] = jnp.zeros_like(c_ref)

    _lstm_steps(xs_ref, wi_ref, wh_ref, b_ref, ys_ref, h_ref, c_ref)


def _lstm_body_cont(ys_in_ref, xs_ref, wi_ref, wh_ref, b_ref, h0_ref, c0_ref,
                    ys_ref, h_ref, c_ref):
    del ys_in_ref
    gi = pl.program_id(0)

    @pl.when(gi == 0)
    def _init():
        h_ref[...] = h0_ref[...]
        c_ref[...] = c0_ref[...]

    _lstm_steps(xs_ref, wi_ref, wh_ref, b_ref, ys_ref, h_ref, c_ref)


def _lstm_steps(xs_ref, wi_ref, wh_ref, b_ref, ys_ref, h_ref, c_ref):
    h = h_ref[...]
    c = c_ref[...]
    wi = wi_ref[...]
    wh = wh_ref[...]
    bias = b_ref[...]
    for k in range(S_PER_BLOCK):
        x = xs_ref[k * B:(k + 1) * B, 0:EMBED]
        gates = (
            lax.dot_general(wi, x, (((1,), (1,)), ((), ())),
                            preferred_element_type=jnp.float32)
            + lax.dot_general(wh, h, (((1,), (0,)), ((), ())),
                              preferred_element_type=jnp.float32)
            + bias
        )
        gi_ = jax.nn.sigmoid(gates[0 * HIDDEN:1 * HIDDEN, :])
        gf = jax.nn.sigmoid(gates[1 * HIDDEN:2 * HIDDEN, :])
        gg = jnp.tanh(gates[2 * HIDDEN:3 * HIDDEN, :])
        go = jax.nn.sigmoid(gates[3 * HIDDEN:4 * HIDDEN, :])
        c = gf * c + gi_ * gg
        h = go * jnp.tanh(c)
        ys_ref[k * HIDDEN:(k + 1) * HIDDEN, :] = h
    h_ref[...] = h
    c_ref[...] = c


_WEIGHT_SPECS = [
    pl.BlockSpec((4 * HIDDEN, EMBED), lambda i: (0, 0)),
    pl.BlockSpec((4 * HIDDEN, HIDDEN), lambda i: (0, 0)),
    pl.BlockSpec((4 * HIDDEN, 1), lambda i: (0, 0)),
]
_HC_SPEC = pl.BlockSpec((HIDDEN, B), lambda i: (0, 0))


def _tc_lstm_first(xs1, wi, wh, bias, t_part):
    nblk = t_part // S_PER_BLOCK
    return pl.pallas_call(
        _lstm_body_first,
        grid=(nblk,),
        in_specs=[
            pl.BlockSpec((S_PER_BLOCK * B, 2 * EMBED), lambda i: (i, 0)),
            *_WEIGHT_SPECS,
        ],
        out_specs=[
            pl.BlockSpec((S_PER_BLOCK * HIDDEN, B), lambda i: (i, 0)),
            _HC_SPEC,
            _HC_SPEC,
        ],
        out_shape=[
            jax.ShapeDtypeStruct((T * HIDDEN, B), jnp.float32),
            jax.ShapeDtypeStruct((HIDDEN, B), jnp.float32),
            jax.ShapeDtypeStruct((HIDDEN, B), jnp.float32),
        ],
        compiler_params=pltpu.CompilerParams(
            dimension_semantics=("arbitrary",),
        ),
    )(xs1, wi, wh, bias)


def _tc_lstm_cont(ys_prev, xs2, wi, wh, bias, h0, c0, t_part, t0):
    nblk = t_part // S_PER_BLOCK
    blk0 = t0 // S_PER_BLOCK
    return pl.pallas_call(
        _lstm_body_cont,
        grid=(nblk,),
        in_specs=[
            pl.BlockSpec(memory_space=pl.ANY),
            pl.BlockSpec((S_PER_BLOCK * B, 2 * EMBED), lambda i: (i, 0)),
            *_WEIGHT_SPECS,
            _HC_SPEC,
            _HC_SPEC,
        ],
        out_specs=[
            pl.BlockSpec((S_PER_BLOCK * HIDDEN, B),
                         lambda i: (i + blk0, 0)),
            _HC_SPEC,
            _HC_SPEC,
        ],
        out_shape=[
            jax.ShapeDtypeStruct((T * HIDDEN, B), jnp.float32),
            jax.ShapeDtypeStruct((HIDDEN, B), jnp.float32),
            jax.ShapeDtypeStruct((HIDDEN, B), jnp.float32),
        ],
        input_output_aliases={0: 0},
        compiler_params=pltpu.CompilerParams(
            dimension_semantics=("arbitrary",),
        ),
    )(ys_prev, xs2, wi, wh, bias, h0, c0)


def kernel(src, table, W_ih, W_hh, b_ih, b_hh):
    idx_flat = src.reshape(-1)
    t128 = _tc_transpose_pad(jnp.transpose(table))

    starts = [sum(T_PARTS[:i]) for i in range(len(T_PARTS))]
    xs_parts = [
        _sc_gather(t128, idx_flat, t0 * B, tp * B)
        for t0, tp in zip(starts, T_PARTS)
    ]

    bias = (b_ih + b_hh)[:, None]

    ysT, hT, cT = _tc_lstm_first(xs_parts[0], W_ih, W_hh, bias, T_PARTS[0])
    for i in range(1, len(T_PARTS)):
        ysT, hT, cT = _tc_lstm_cont(ysT, xs_parts[i], W_ih, W_hh, bias,
                                    hT, cT, T_PARTS[i], starts[i])
    outputs = ysT.reshape(T, HIDDEN, B).transpose(2, 0, 1)
    hidden = hT.transpose(1, 0)[None]
    cell = cT.transpose(1, 0)[None]
    return (outputs, hidden, cell)


# speedup vs baseline: 2.7084x; 1.0152x over previous
import functools

import jax
import jax.numpy as jnp
from jax import lax
from jax.experimental import pallas as pl
from jax.experimental.pallas import tpu as pltpu
from jax.experimental.pallas import tpu_sc as plsc

VOCAB = 1000000
EMBED = 64
HIDDEN = 64
B = 1024
T = 200
T_PARTS = (48, 48, 48, 56)

NUM_CORES = 2
NUM_SUBCORES = 16
NUM_WORKERS = NUM_CORES * NUM_SUBCORES
LANES = 16

GATHER_CHUNK = 128

S_PER_BLOCK = 8


TROWS = 32768


def _transpose_body(tt_ref, out_ref):
    out_ref[:, 0:EMBED] = jnp.swapaxes(tt_ref[...], 0, 1)


def _tc_transpose_pad(tableT):
    vpad = ((VOCAB + TROWS - 1) // TROWS) * TROWS
    nblk = vpad // TROWS
    return pl.pallas_call(
        _transpose_body,
        grid=(nblk,),
        in_specs=[pl.BlockSpec((EMBED, TROWS), lambda i: (0, i))],
        out_specs=pl.BlockSpec((TROWS, 2 * EMBED), lambda i: (i, 0)),
        out_shape=jax.ShapeDtypeStruct((vpad, 2 * EMBED), jnp.float32),
        compiler_params=pltpu.CompilerParams(
            dimension_semantics=("arbitrary",),
        ),
    )(tableT)


def _sc_gather(table, idx_flat, q0, n):
    per_w = n // NUM_WORKERS
    chunks = per_w // GATHER_CHUNK
    assert per_w * NUM_WORKERS == n and chunks * GATHER_CHUNK == per_w
    assert chunks % 2 == 0 and q0 % GATHER_CHUNK == 0

    mesh = plsc.VectorSubcoreMesh(core_axis_name="c", subcore_axis_name="s")

    @functools.partial(
        pl.kernel,
        out_type=jax.ShapeDtypeStruct((n, 2 * EMBED), jnp.float32),
        mesh=mesh,
        scratch_types=[
            pltpu.VMEM((2, GATHER_CHUNK), jnp.int32),
            pltpu.VMEM((2, GATHER_CHUNK), jnp.int32),
            pltpu.VMEM((2, GATHER_CHUNK, 2 * EMBED), jnp.float32),
            pltpu.SemaphoreType.DMA((2,)),
            pltpu.SemaphoreType.DMA((2,)),
        ],
        compiler_params=pltpu.CompilerParams(use_tc_tiling_on_sc=False),
    )
    def gather_kernel(table_hbm, idx_hbm, out_hbm, pos_v, idxg_v, rows_v,
                      sem_i, sem_r):
        wid = lax.axis_index("s") * NUM_CORES + lax.axis_index("c")
        base_w = wid * per_w
        lane = lax.iota(jnp.int32, LANES)

        def fire_idx(c, buf):
            base = base_w + c * GATHER_CHUNK
            for j in range(GATHER_CHUNK // LANES):
                q = q0 + base + j * LANES + lane
                t = lax.shift_right_logical(q, 10)
                b = lax.bitwise_and(q, B - 1)
                pos_v[buf, pl.ds(j * LANES, LANES)] = b * T + t
            pltpu.async_copy(idx_hbm.at[pos_v.at[buf]], idxg_v.at[buf],
                             sem_i.at[buf])

        def wait_idx(buf):
            pltpu.make_async_copy(idx_hbm.at[pl.ds(0, GATHER_CHUNK)],
                                  idxg_v.at[buf], sem_i.at[buf]).wait()

        def fire_rows(buf):
            pltpu.async_copy(table_hbm.at[idxg_v.at[buf]], rows_v.at[buf],
                             sem_r.at[buf])

        def wait_rows(buf):
            pltpu.make_async_copy(table_hbm.at[pl.ds(0, GATHER_CHUNK)],
                                  rows_v.at[buf], sem_r.at[buf]).wait()

        def write_out(c, buf):
            base = base_w + c * GATHER_CHUNK
            pltpu.sync_copy(rows_v.at[buf],
                            out_hbm.at[pl.ds(base, GATHER_CHUNK)])

        fire_idx(0, 0)
        wait_idx(0)
        fire_rows(0)
        fire_idx(1, 1)

        @pl.loop(0, chunks, step=2)
        def _chunk(c):
            wait_idx(1)
            fire_rows(1)
            wait_rows(0)
            write_out(c, 0)

            @pl.when(c + 2 < chunks)
            def _next_even():
                fire_idx(c + 2, 0)
                wait_idx(0)
                fire_rows(0)

            wait_rows(1)
            write_out(c + 1, 1)

            @pl.when(c + 3 < chunks)
            def _next_odd():
                fire_idx(c + 3, 1)

    return gather_kernel(table, idx_flat)


def _lstm_body_first(xs_ref, wi_ref, wh_ref, b_ref, ys_ref, h_ref, c_ref):
    gi = pl.program_id(0)

    @pl.when(gi == 0)
    def _init():
        h_ref[...] = jnp.zeros_like(h_ref)
        c_ref[...] = jnp.zeros_like(c_ref)

    _lstm_steps(xs_ref, wi_ref, wh_ref, b_ref, ys_ref, h_ref, c_ref)


def _lstm_body_cont(ys_in_ref, xs_ref, wi_ref, wh_ref, b_ref, h0_ref, c0_ref,
                    ys_ref, h_ref, c_ref):
    del ys_in_ref
    gi = pl.program_id(0)

    @pl.when(gi == 0)
    def _init():
        h_ref[...] = h0_ref[...]
        c_ref[...] = c0_ref[...]

    _lstm_steps(xs_ref, wi_ref, wh_ref, b_ref, ys_ref, h_ref, c_ref)


def _lstm_steps(xs_ref, wi_ref, wh_ref, b_ref, ys_ref, h_ref, c_ref):
    h = h_ref[...]
    c = c_ref[...]
    wi = wi_ref[...]
    wh = wh_ref[...]
    bias = b_ref[...]
    for k in range(S_PER_BLOCK):
        x = xs_ref[k * B:(k + 1) * B, 0:EMBED]
        gates = (
            lax.dot_general(wi, x, (((1,), (1,)), ((), ())),
                            preferred_element_type=jnp.float32)
            + lax.dot_general(wh, h, (((1,), (0,)), ((), ())),
                              preferred_element_type=jnp.float32)
            + bias
        )
        gi_ = jax.nn.sigmoid(gates[0 * HIDDEN:1 * HIDDEN, :])
        gf = jax.nn.sigmoid(gates[1 * HIDDEN:2 * HIDDEN, :])
        gg = jnp.tanh(gates[2 * HIDDEN:3 * HIDDEN, :])
        go = jax.nn.sigmoid(gates[3 * HIDDEN:4 * HIDDEN, :])
        c = gf * c + gi_ * gg
        h = go * jnp.tanh(c)
        ys_ref[k * HIDDEN:(k + 1) * HIDDEN, :] = h
    h_ref[...] = h
    c_ref[...] = c


_WEIGHT_SPECS = [
    pl.BlockSpec((4 * HIDDEN, EMBED), lambda i: (0, 0)),
    pl.BlockSpec((4 * HIDDEN, HIDDEN), lambda i: (0, 0)),
    pl.BlockSpec((4 * HIDDEN, 1), lambda i: (0, 0)),
]
_HC_SPEC = pl.BlockSpec((HIDDEN, B), lambda i: (0, 0))


def _tc_lstm_first(xs1, wi, wh, bias, t_part):
    nblk = t_part // S_PER_BLOCK
    return pl.pallas_call(
        _lstm_body_first,
        grid=(nblk,),
        in_specs=[
            pl.BlockSpec((S_PER_BLOCK * B, 2 * EMBED), lambda i: (i, 0)),
            *_WEIGHT_SPECS,
        ],
        out_specs=[
            pl.BlockSpec((S_PER_BLOCK * HIDDEN, B), lambda i: (i, 0)),
            _HC_SPEC,
            _HC_SPEC,
        ],
        out_shape=[
            jax.ShapeDtypeStruct((T * HIDDEN, B), jnp.float32),
            jax.ShapeDtypeStruct((HIDDEN, B), jnp.float32),
            jax.ShapeDtypeStruct((HIDDEN, B), jnp.float32),
        ],
        compiler_params=pltpu.CompilerParams(
            dimension_semantics=("arbitrary",),
        ),
    )(xs1, wi, wh, bias)


def _tc_lstm_cont(ys_prev, xs2, wi, wh, bias, h0, c0, t_part, t0):
    nblk = t_part // S_PER_BLOCK
    blk0 = t0 // S_PER_BLOCK
    return pl.pallas_call(
        _lstm_body_cont,
        grid=(nblk,),
        in_specs=[
            pl.BlockSpec(memory_space=pl.ANY),
            pl.BlockSpec((S_PER_BLOCK * B, 2 * EMBED), lambda i: (i, 0)),
            *_WEIGHT_SPECS,
            _HC_SPEC,
            _HC_SPEC,
        ],
        out_specs=[
            pl.BlockSpec((S_PER_BLOCK * HIDDEN, B),
                         lambda i: (i + blk0, 0)),
            _HC_SPEC,
            _HC_SPEC,
        ],
        out_shape=[
            jax.ShapeDtypeStruct((T * HIDDEN, B), jnp.float32),
            jax.ShapeDtypeStruct((HIDDEN, B), jnp.float32),
            jax.ShapeDtypeStruct((HIDDEN, B), jnp.float32),
        ],
        input_output_aliases={0: 0},
        compiler_params=pltpu.CompilerParams(
            dimension_semantics=("arbitrary",),
        ),
    )(ys_prev, xs2, wi, wh, bias, h0, c0)


def kernel(src, table, W_ih, W_hh, b_ih, b_hh):
    idx_flat = src.reshape(-1)
    t128 = _tc_transpose_pad(jnp.transpose(table))

    starts = [sum(T_PARTS[:i]) for i in range(len(T_PARTS))]
    xs_parts = [
        _sc_gather(t128, idx_flat, t0 * B, tp * B)
        for t0, tp in zip(starts, T_PARTS)
    ]

    bias = (b_ih + b_hh)[:, None]

    ysT, hT, cT = _tc_lstm_first(xs_parts[0], W_ih, W_hh, bias, T_PARTS[0])
    for i in range(1, len(T_PARTS)):
        ysT, hT, cT = _tc_lstm_cont(ysT, xs_parts[i], W_ih, W_hh, bias,
                                    hT, cT, T_PARTS[i], starts[i])
    outputs = ysT.reshape(T, HIDDEN, B).transpose(2, 0, 1)
    hidden = hT.transpose(1, 0)[None]
    cell = cT.transpose(1, 0)[None]
    return (outputs, hidden, cell)
